# Initial kernel scaffold; baseline (speedup 1.0000x reference)
import jax
import jax.numpy as jnp
from jax.experimental import pallas as pl


def kernel(atomic_numbers, edge_index, edge_features, batch, params):
    raise NotImplementedError("write your pallas kernel here")


# speedup vs baseline: 1.9208x; 1.9208x over previous
import functools

import jax
import jax.numpy as jnp
from jax import lax
from jax.experimental import pallas as pl
from jax.experimental.pallas import tpu as pltpu
from jax.experimental.pallas import tpu_sc as plsc

N = 10000
E = 320000
H = 128
ED = 10
B = 64
NELEM = 92

NCORES = 2
NSUB = 16
NW = NCORES * NSUB
CHUNK = 128
EPW = 10240
EPAD = NW * EPW
NCH = EPW // CHUNK
NACC = 10240
RPT = NACC // NSUB
BE = 2048

@functools.cache
def _sc_mesh():
  return plsc.VectorSubcoreMesh(
      core_axis_name="c", subcore_axis_name="s",
      num_cores=NCORES, num_subcores=NSUB)


def _sc_gather(x, srcp, dstp):
  @functools.partial(
      pl.kernel,
      out_type=[jax.ShapeDtypeStruct((EPAD, H), jnp.float32),
                jax.ShapeDtypeStruct((EPAD, H), jnp.float32)],
      mesh=_sc_mesh(),
      scratch_types=[pltpu.VMEM((EPW,), jnp.int32),
                     pltpu.VMEM((EPW,), jnp.int32),
                     pltpu.VMEM((CHUNK, H), jnp.float32),
                     pltpu.VMEM((CHUNK, H), jnp.float32),
                     pltpu.SemaphoreType.DMA,
                     pltpu.SemaphoreType.DMA],
  )
  def k(x_hbm, src_hbm, dst_hbm, sf_hbm, df_hbm, sidx, didx, srows, drows,
        sem_s, sem_d):
    wid = lax.axis_index("s") * NCORES + lax.axis_index("c")
    base = wid * EPW
    pltpu.sync_copy(src_hbm.at[pl.ds(base, EPW)], sidx)
    pltpu.sync_copy(dst_hbm.at[pl.ds(base, EPW)], didx)

    def body(j, carry):
      off = j * CHUNK
      cs = pltpu.async_copy(x_hbm.at[sidx.at[pl.ds(off, CHUNK)]], srows, sem_s)
      cd = pltpu.async_copy(x_hbm.at[didx.at[pl.ds(off, CHUNK)]], drows, sem_d)
      cs.wait()
      pltpu.sync_copy(srows, sf_hbm.at[pl.ds(base + off, CHUNK)])
      cd.wait()
      pltpu.sync_copy(drows, df_hbm.at[pl.ds(base + off, CHUNK)])
      return carry

    lax.fori_loop(0, NCH, body, 0)

  return k(x, srcp, dstp)


def _sc_scatter(msgs, dst2d, zrows):
  @functools.partial(
      pl.kernel,
      out_type=jax.ShapeDtypeStruct((NCORES, NACC, H), jnp.float32),
      mesh=_sc_mesh(),
      scratch_types=[pltpu.VMEM((NCH, CHUNK), jnp.int32),
                     pltpu.VMEM((CHUNK, H), jnp.float32),
                     pltpu.VMEM_SHARED((NACC, H), jnp.float32)],
  )
  def k(msgs_hbm, dst_hbm, z_hbm, out_hbm, didx, rows, acc):
    c = lax.axis_index("c")
    s = lax.axis_index("s")
    wid = s * NCORES + c
    pltpu.sync_copy(z_hbm, rows)

    def zbody(i, carry):
      pltpu.sync_copy(rows, acc.at[pl.ds(s * RPT + i * CHUNK, CHUNK)])
      return carry

    lax.fori_loop(0, RPT // CHUNK, zbody, 0)
    plsc.subcore_barrier()

    pltpu.sync_copy(dst_hbm.at[pl.ds(wid * NCH, NCH)], didx)
    ebase = wid * EPW

    def body(j, carry):
      pltpu.sync_copy(msgs_hbm.at[pl.ds(ebase + j * CHUNK, CHUNK)], rows)
      pltpu.sync_copy(rows, acc.at[didx.at[j]], add=True)
      return carry

    lax.fori_loop(0, NCH, body, 0)
    plsc.subcore_barrier()

    def wbody(i, carry):
      off = s * RPT + i * CHUNK
      pltpu.sync_copy(acc.at[pl.ds(off, CHUNK)], rows)
      pltpu.sync_copy(rows, out_hbm.at[c].at[pl.ds(off, CHUNK)])
      return carry

    lax.fori_loop(0, RPT // CHUNK, wbody, 0)

  return k(msgs, dst2d, zrows)


def _embed_body(an_ref, tab_ref, out_ref):
  oh = (an_ref[...] == lax.broadcasted_iota(jnp.int32, (1, 128), 1))
  out_ref[...] = jnp.dot(oh.astype(jnp.float32), tab_ref[...],
                         preferred_element_type=jnp.float32)


def _tc_embed(an2d, tab):
  return pl.pallas_call(
      _embed_body,
      out_shape=jax.ShapeDtypeStruct((N, H), jnp.float32),
  )(an2d, tab)


def _edge_body(sf_ref, df_ref, ef_ref, ws_ref, wd_ref, wef_ref, bf_ref,
               w2_ref, b2_ref, out_ref):
  sfb = sf_ref[...]
  h = jnp.dot(sfb, ws_ref[...], preferred_element_type=jnp.float32)
  h = h + jnp.dot(df_ref[...], wd_ref[...], preferred_element_type=jnp.float32)
  h = h + jnp.dot(ef_ref[...], wef_ref[...], preferred_element_type=jnp.float32)
  h = jax.nn.relu(h + bf_ref[...])
  ew = jnp.dot(h, w2_ref[...], preferred_element_type=jnp.float32) + b2_ref[...]
  out_ref[...] = sfb * jax.nn.sigmoid(ew)


def _tc_edge_mlp(sf, df, efp, ws, wd, wef, bf, w2, b2):
  nb = EPAD // BE
  blk = lambda i: (i, 0)
  fix = lambda i: (0, 0)
  return pl.pallas_call(
      _edge_body,
      grid=(nb,),
      in_specs=[pl.BlockSpec((BE, H), blk),
                pl.BlockSpec((BE, H), blk),
                pl.BlockSpec((BE, 16), blk),
                pl.BlockSpec((H, H), fix),
                pl.BlockSpec((H, H), fix),
                pl.BlockSpec((16, H), fix),
                pl.BlockSpec((1, H), fix),
                pl.BlockSpec((H, H), fix),
                pl.BlockSpec((1, H), fix)],
      out_specs=pl.BlockSpec((BE, H), blk),
      out_shape=jax.ShapeDtypeStruct((EPAD, H), jnp.float32),
  )(sf, df, efp, ws, wd, wef, bf, w2, b2)


def _node_body(x_ref, a_ref, w1a_ref, w1b_ref, b1_ref, w2_ref, b2_ref,
               g_ref, bb_ref, out_ref):
  x = x_ref[...]
  a = a_ref[0, :N, :] + a_ref[1, :N, :]
  t = jnp.dot(x, w1a_ref[...], preferred_element_type=jnp.float32)
  t = t + jnp.dot(a, w1b_ref[...], preferred_element_type=jnp.float32)
  t = jax.nn.relu(t + b1_ref[...])
  u = jnp.dot(t, w2_ref[...], preferred_element_type=jnp.float32) + b2_ref[...]
  mu = jnp.mean(u, axis=0, keepdims=True)
  d = u - mu
  var = jnp.mean(d * d, axis=0, keepdims=True)
  un = d / jnp.sqrt(var + 1e-5) * g_ref[...] + bb_ref[...]
  out_ref[...] = x + un


def _tc_node(x, agg2, w1a, w1b, b1, w2, b2, g, bb):
  return pl.pallas_call(
      _node_body,
      out_shape=jax.ShapeDtypeStruct((N, H), jnp.float32),
  )(x, agg2, w1a, w1b, b1, w2, b2, g, bb)


def _readout_body(x_ref, bt_ref, w1_ref, b1_ref, w2_ref, b2_ref, out_ref):
  bt = bt_ref[...]
  oh = (bt == lax.broadcasted_iota(jnp.int32, (B, 1), 0)).astype(jnp.float32)
  pooled = jnp.dot(oh, x_ref[...], preferred_element_type=jnp.float32)
  counts = jnp.sum(oh, axis=1, keepdims=True)
  pooled = pooled / jnp.clip(counts, 1.0, None)
  t = jax.nn.relu(jnp.dot(pooled, w1_ref[...],
                          preferred_element_type=jnp.float32) + b1_ref[...])
  emb = jnp.dot(t, w2_ref[...], preferred_element_type=jnp.float32) + b2_ref[...]
  nrm = jnp.sqrt(jnp.sum(emb * emb, axis=1, keepdims=True))
  out_ref[...] = emb / jnp.maximum(nrm, 1e-12)


def _tc_readout(x, brow, w1, b1, w2, b2):
  return pl.pallas_call(
      _readout_body,
      out_shape=jax.ShapeDtypeStruct((B, H), jnp.float32),
  )(x, brow, w1, b1, w2, b2)


def kernel(atomic_numbers, edge_index, edge_features, batch, params):
  f32 = jnp.float32
  src = edge_index[0].astype(jnp.int32)
  dst = edge_index[1].astype(jnp.int32)
  srcp = jnp.pad(src, (0, EPAD - E))
  dstp = jnp.pad(dst, (0, EPAD - E), constant_values=N)
  dst2d = dstp.reshape(NW * NCH, CHUNK)
  efp = jnp.pad(edge_features.astype(f32), ((0, EPAD - E), (0, 16 - ED)))
  tab = jnp.pad(params["elem_table"].astype(f32), ((0, 128 - NELEM), (0, 0)))
  an2d = (atomic_numbers.astype(jnp.int32) - 1).reshape(N, 1)
  zrows = jnp.zeros((CHUNK, H), f32)

  x = _tc_embed(an2d, tab)

  wep = jnp.pad(params["edge_embed"]["w"].astype(f32), ((0, 16 - ED), (0, 0)))
  be = params["edge_embed"]["b"].astype(f32)

  for c in params["convs"]:
    w1 = c["edge1"]["w"].astype(f32)
    ws, wd, w1e = w1[:H], w1[H:2 * H], w1[2 * H:]
    wef = wep @ w1e
    bf = (be @ w1e + c["edge1"]["b"]).reshape(1, H)
    sf, df = _sc_gather(x, srcp, dstp)
    msgs = _tc_edge_mlp(sf, df, efp, ws, wd, wef, bf,
                        c["edge2"]["w"].astype(f32),
                        c["edge2"]["b"].reshape(1, H).astype(f32))
    agg2 = _sc_scatter(msgs, dst2d, zrows)
    x = _tc_node(x, agg2,
                 c["node1"]["w"][:H].astype(f32),
                 c["node1"]["w"][H:].astype(f32),
                 c["node1"]["b"].reshape(1, H).astype(f32),
                 c["node2"]["w"].astype(f32),
                 c["node2"]["b"].reshape(1, H).astype(f32),
                 c["bn_g"].reshape(1, H).astype(f32),
                 c["bn_b"].reshape(1, H).astype(f32))

  brow = batch.astype(jnp.int32).reshape(1, N)
  return _tc_readout(x, brow,
                     params["readout1"]["w"].astype(f32),
                     params["readout1"]["b"].reshape(1, H).astype(f32),
                     params["readout2"]["w"].astype(f32),
                     params["readout2"]["b"].reshape(1, H).astype(f32))


# speedup vs baseline: 2.0495x; 1.0670x over previous
import functools

import jax
import jax.numpy as jnp
from jax import lax
from jax.experimental import pallas as pl
from jax.experimental.pallas import tpu as pltpu
from jax.experimental.pallas import tpu_sc as plsc

N = 10000
E = 320000
H = 128
ED = 10
B = 64
NELEM = 92

NCORES = 2
NSUB = 16
NW = NCORES * NSUB
CHUNK = 128
EPW = 10240
EPAD = NW * EPW
NCH = EPW // CHUNK
NACC = 10240
RPT = NACC // NSUB
BE = 2048

@functools.cache
def _sc_mesh():
  return plsc.VectorSubcoreMesh(
      core_axis_name="c", subcore_axis_name="s",
      num_cores=NCORES, num_subcores=NSUB)


def _sc_gather(x, srcp, dstp):
  @functools.partial(
      pl.kernel,
      out_type=[jax.ShapeDtypeStruct((EPAD, H), jnp.float32),
                jax.ShapeDtypeStruct((EPAD, H), jnp.float32)],
      mesh=_sc_mesh(),
      scratch_types=[pltpu.VMEM((EPW,), jnp.int32),
                     pltpu.VMEM((EPW,), jnp.int32),
                     pltpu.VMEM((2, CHUNK, H), jnp.float32),
                     pltpu.VMEM((2, CHUNK, H), jnp.float32),
                     [pltpu.SemaphoreType.DMA] * 2,
                     [pltpu.SemaphoreType.DMA] * 2,
                     [pltpu.SemaphoreType.DMA] * 2,
                     [pltpu.SemaphoreType.DMA] * 2],
  )
  def k(x_hbm, src_hbm, dst_hbm, sf_hbm, df_hbm, sidx, didx, srows, drows,
        gsem_s, gsem_d, wsem_s, wsem_d):
    wid = lax.axis_index("s") * NCORES + lax.axis_index("c")
    base = wid * EPW
    pltpu.sync_copy(src_hbm.at[pl.ds(base, EPW)], sidx)
    pltpu.sync_copy(dst_hbm.at[pl.ds(base, EPW)], didx)

    def body(g, carry):
      for k in range(2):
        off = (2 * g + k) * CHUNK

        @pl.when(g > 0)
        def _drain():
          pltpu.make_async_copy(
              srows.at[k], sf_hbm.at[pl.ds(base, CHUNK)], wsem_s[k]).wait()
          pltpu.make_async_copy(
              drows.at[k], df_hbm.at[pl.ds(base, CHUNK)], wsem_d[k]).wait()

        pltpu.async_copy(x_hbm.at[sidx.at[pl.ds(off, CHUNK)]],
                         srows.at[k], gsem_s[k])
        pltpu.async_copy(x_hbm.at[didx.at[pl.ds(off, CHUNK)]],
                         drows.at[k], gsem_d[k])
      for k in range(2):
        off = (2 * g + k) * CHUNK
        pltpu.make_async_copy(x_hbm.at[sidx.at[pl.ds(off, CHUNK)]],
                              srows.at[k], gsem_s[k]).wait()
        pltpu.async_copy(srows.at[k], sf_hbm.at[pl.ds(base + off, CHUNK)],
                         wsem_s[k])
        pltpu.make_async_copy(x_hbm.at[didx.at[pl.ds(off, CHUNK)]],
                              drows.at[k], gsem_d[k]).wait()
        pltpu.async_copy(drows.at[k], df_hbm.at[pl.ds(base + off, CHUNK)],
                         wsem_d[k])
      return carry

    lax.fori_loop(0, NCH // 2, body, 0)
    for k in range(2):
      pltpu.make_async_copy(
          srows.at[k], sf_hbm.at[pl.ds(base, CHUNK)], wsem_s[k]).wait()
      pltpu.make_async_copy(
          drows.at[k], df_hbm.at[pl.ds(base, CHUNK)], wsem_d[k]).wait()

  return k(x, srcp, dstp)


def _sc_scatter(msgs, dst2d, zrows):
  @functools.partial(
      pl.kernel,
      out_type=jax.ShapeDtypeStruct((NCORES, NACC, H), jnp.float32),
      mesh=_sc_mesh(),
      scratch_types=[pltpu.VMEM((NCH, CHUNK), jnp.int32),
                     pltpu.VMEM((CHUNK, H), jnp.float32),
                     pltpu.VMEM_SHARED((NACC, H), jnp.float32)],
  )
  def k(msgs_hbm, dst_hbm, z_hbm, out_hbm, didx, rows, acc):
    c = lax.axis_index("c")
    s = lax.axis_index("s")
    wid = s * NCORES + c
    pltpu.sync_copy(z_hbm, rows)

    def zbody(i, carry):
      pltpu.sync_copy(rows, acc.at[pl.ds(s * RPT + i * CHUNK, CHUNK)])
      return carry

    lax.fori_loop(0, RPT // CHUNK, zbody, 0)
    plsc.subcore_barrier()

    pltpu.sync_copy(dst_hbm.at[pl.ds(wid * NCH, NCH)], didx)
    ebase = wid * EPW

    def body(j, carry):
      pltpu.sync_copy(msgs_hbm.at[pl.ds(ebase + j * CHUNK, CHUNK)], rows)
      pltpu.sync_copy(rows, acc.at[didx.at[j]], add=True)
      return carry

    lax.fori_loop(0, NCH, body, 0)
    plsc.subcore_barrier()

    def wbody(i, carry):
      off = s * RPT + i * CHUNK
      pltpu.sync_copy(acc.at[pl.ds(off, CHUNK)], rows)
      pltpu.sync_copy(rows, out_hbm.at[c].at[pl.ds(off, CHUNK)])
      return carry

    lax.fori_loop(0, RPT // CHUNK, wbody, 0)

  return k(msgs, dst2d, zrows)


def _embed_body(an_ref, tab_ref, out_ref):
  oh = (an_ref[...] == lax.broadcasted_iota(jnp.int32, (1, 128), 1))
  out_ref[...] = jnp.dot(oh.astype(jnp.float32), tab_ref[...],
                         preferred_element_type=jnp.float32)


def _tc_embed(an2d, tab):
  return pl.pallas_call(
      _embed_body,
      out_shape=jax.ShapeDtypeStruct((N, H), jnp.float32),
  )(an2d, tab)


def _edge_body(sf_ref, df_ref, ef_ref, ws_ref, wd_ref, wef_ref, bf_ref,
               w2_ref, b2_ref, out_ref):
  sfb = sf_ref[...]
  h = jnp.dot(sfb, ws_ref[...], preferred_element_type=jnp.float32)
  h = h + jnp.dot(df_ref[...], wd_ref[...], preferred_element_type=jnp.float32)
  h = h + jnp.dot(ef_ref[...], wef_ref[...], preferred_element_type=jnp.float32)
  h = jax.nn.relu(h + bf_ref[...])
  ew = jnp.dot(h, w2_ref[...], preferred_element_type=jnp.float32) + b2_ref[...]
  out_ref[...] = sfb * jax.nn.sigmoid(ew)


def _tc_edge_mlp(sf, df, efp, ws, wd, wef, bf, w2, b2):
  nb = EPAD // BE
  blk = lambda i: (i, 0)
  fix = lambda i: (0, 0)
  return pl.pallas_call(
      _edge_body,
      grid=(nb,),
      in_specs=[pl.BlockSpec((BE, H), blk),
                pl.BlockSpec((BE, H), blk),
                pl.BlockSpec((BE, 16), blk),
                pl.BlockSpec((H, H), fix),
                pl.BlockSpec((H, H), fix),
                pl.BlockSpec((16, H), fix),
                pl.BlockSpec((1, H), fix),
                pl.BlockSpec((H, H), fix),
                pl.BlockSpec((1, H), fix)],
      out_specs=pl.BlockSpec((BE, H), blk),
      out_shape=jax.ShapeDtypeStruct((EPAD, H), jnp.float32),
  )(sf, df, efp, ws, wd, wef, bf, w2, b2)


def _node_body(x_ref, a_ref, w1a_ref, w1b_ref, b1_ref, w2_ref, b2_ref,
               g_ref, bb_ref, out_ref):
  x = x_ref[...]
  a = a_ref[0, :N, :] + a_ref[1, :N, :]
  t = jnp.dot(x, w1a_ref[...], preferred_element_type=jnp.float32)
  t = t + jnp.dot(a, w1b_ref[...], preferred_element_type=jnp.float32)
  t = jax.nn.relu(t + b1_ref[...])
  u = jnp.dot(t, w2_ref[...], preferred_element_type=jnp.float32) + b2_ref[...]
  mu = jnp.mean(u, axis=0, keepdims=True)
  d = u - mu
  var = jnp.mean(d * d, axis=0, keepdims=True)
  un = d / jnp.sqrt(var + 1e-5) * g_ref[...] + bb_ref[...]
  out_ref[...] = x + un


def _tc_node(x, agg2, w1a, w1b, b1, w2, b2, g, bb):
  return pl.pallas_call(
      _node_body,
      out_shape=jax.ShapeDtypeStruct((N, H), jnp.float32),
  )(x, agg2, w1a, w1b, b1, w2, b2, g, bb)


def _readout_body(x_ref, bt_ref, w1_ref, b1_ref, w2_ref, b2_ref, out_ref):
  bt = bt_ref[...]
  oh = (bt == lax.broadcasted_iota(jnp.int32, (B, 1), 0)).astype(jnp.float32)
  pooled = jnp.dot(oh, x_ref[...], preferred_element_type=jnp.float32)
  counts = jnp.sum(oh, axis=1, keepdims=True)
  pooled = pooled / jnp.clip(counts, 1.0, None)
  t = jax.nn.relu(jnp.dot(pooled, w1_ref[...],
                          preferred_element_type=jnp.float32) + b1_ref[...])
  emb = jnp.dot(t, w2_ref[...], preferred_element_type=jnp.float32) + b2_ref[...]
  nrm = jnp.sqrt(jnp.sum(emb * emb, axis=1, keepdims=True))
  out_ref[...] = emb / jnp.maximum(nrm, 1e-12)


def _tc_readout(x, brow, w1, b1, w2, b2):
  return pl.pallas_call(
      _readout_body,
      out_shape=jax.ShapeDtypeStruct((B, H), jnp.float32),
  )(x, brow, w1, b1, w2, b2)


def kernel(atomic_numbers, edge_index, edge_features, batch, params):
  f32 = jnp.float32
  src = edge_index[0].astype(jnp.int32)
  dst = edge_index[1].astype(jnp.int32)
  srcp = jnp.pad(src, (0, EPAD - E))
  dstp = jnp.pad(dst, (0, EPAD - E), constant_values=N)
  dst2d = dstp.reshape(NW * NCH, CHUNK)
  efp = jnp.pad(edge_features.astype(f32), ((0, EPAD - E), (0, 16 - ED)))
  tab = jnp.pad(params["elem_table"].astype(f32), ((0, 128 - NELEM), (0, 0)))
  an2d = (atomic_numbers.astype(jnp.int32) - 1).reshape(N, 1)
  zrows = jnp.zeros((CHUNK, H), f32)

  x = _tc_embed(an2d, tab)

  wep = jnp.pad(params["edge_embed"]["w"].astype(f32), ((0, 16 - ED), (0, 0)))
  be = params["edge_embed"]["b"].astype(f32)

  for c in params["convs"]:
    w1 = c["edge1"]["w"].astype(f32)
    ws, wd, w1e = w1[:H], w1[H:2 * H], w1[2 * H:]
    wef = wep @ w1e
    bf = (be @ w1e + c["edge1"]["b"]).reshape(1, H)
    sf, df = _sc_gather(x, srcp, dstp)
    msgs = _tc_edge_mlp(sf, df, efp, ws, wd, wef, bf,
                        c["edge2"]["w"].astype(f32),
                        c["edge2"]["b"].reshape(1, H).astype(f32))
    agg2 = _sc_scatter(msgs, dst2d, zrows)
    x = _tc_node(x, agg2,
                 c["node1"]["w"][:H].astype(f32),
                 c["node1"]["w"][H:].astype(f32),
                 c["node1"]["b"].reshape(1, H).astype(f32),
                 c["node2"]["w"].astype(f32),
                 c["node2"]["b"].reshape(1, H).astype(f32),
                 c["bn_g"].reshape(1, H).astype(f32),
                 c["bn_b"].reshape(1, H).astype(f32))

  brow = batch.astype(jnp.int32).reshape(1, N)
  return _tc_readout(x, brow,
                     params["readout1"]["w"].astype(f32),
                     params["readout1"]["b"].reshape(1, H).astype(f32),
                     params["readout2"]["w"].astype(f32),
                     params["readout2"]["b"].reshape(1, H).astype(f32))


# speedup vs baseline: 3.7289x; 1.8194x over previous
import functools

import jax
import jax.numpy as jnp
from jax import lax
from jax.experimental import pallas as pl
from jax.experimental.pallas import tpu as pltpu
from jax.experimental.pallas import tpu_sc as plsc

N = 10000
E = 320000
H = 128
ED = 10
B = 64
NELEM = 92

NCORES = 2
NSUB = 16
NW = NCORES * NSUB
CHUNK = 128
EPW = 10240
EPAD = NW * EPW
NCH = EPW // CHUNK
NACC = 10240
RPT = NACC // NSUB
BE = 2048
GCH = 64
NGC = EPW // GCH
IB = 16

@functools.cache
def _sc_mesh():
  return plsc.VectorSubcoreMesh(
      core_axis_name="c", subcore_axis_name="s",
      num_cores=NCORES, num_subcores=NSUB)


def _sc_gather(xpad, src2d, dst2d):
  @functools.partial(
      pl.kernel,
      out_type=[jax.ShapeDtypeStruct((EPAD, H), jnp.float32),
                jax.ShapeDtypeStruct((EPAD, H), jnp.float32)],
      mesh=_sc_mesh(),
      scratch_types=[pltpu.VMEM((IB, GCH), jnp.int32),
                     pltpu.VMEM((IB, GCH), jnp.int32),
                     pltpu.VMEM((2, GCH, H), jnp.float32),
                     pltpu.VMEM((2, GCH, H), jnp.float32),
                     pltpu.VMEM_SHARED((NACC, H), jnp.float32),
                     [pltpu.SemaphoreType.DMA] * 2,
                     [pltpu.SemaphoreType.DMA] * 2,
                     [pltpu.SemaphoreType.DMA] * 2,
                     [pltpu.SemaphoreType.DMA] * 2],
  )
  def k(x_hbm, src_hbm, dst_hbm, sf_hbm, df_hbm, sidx, didx, srows, drows,
        xsh, gsem_s, gsem_d, wsem_s, wsem_d):
    s = lax.axis_index("s")
    wid = s * NCORES + lax.axis_index("c")
    base = wid * EPW
    ibase = wid * NGC

    pltpu.sync_copy(x_hbm.at[pl.ds(s * RPT, RPT)], xsh.at[pl.ds(s * RPT, RPT)])
    plsc.subcore_barrier()

    def body(g, carry):
      @pl.when(g % (IB // 2) == 0)
      def _reload():
        blk = g // (IB // 2)
        pltpu.sync_copy(src_hbm.at[pl.ds(ibase + blk * IB, IB)], sidx)
        pltpu.sync_copy(dst_hbm.at[pl.ds(ibase + blk * IB, IB)], didx)

      r0 = (2 * g) % IB
      for k in range(2):
        @pl.when(g > 0)
        def _drain():
          pltpu.make_async_copy(
              srows.at[k], sf_hbm.at[pl.ds(base, GCH)], wsem_s[k]).wait()
          pltpu.make_async_copy(
              drows.at[k], df_hbm.at[pl.ds(base, GCH)], wsem_d[k]).wait()

        pltpu.async_copy(xsh.at[sidx.at[r0 + k]], srows.at[k], gsem_s[k])
        pltpu.async_copy(xsh.at[didx.at[r0 + k]], drows.at[k], gsem_d[k])
      for k in range(2):
        off = (2 * g + k) * GCH
        pltpu.make_async_copy(xsh.at[sidx.at[r0 + k]],
                              srows.at[k], gsem_s[k]).wait()
        pltpu.async_copy(srows.at[k], sf_hbm.at[pl.ds(base + off, GCH)],
                         wsem_s[k])
        pltpu.make_async_copy(xsh.at[didx.at[r0 + k]],
                              drows.at[k], gsem_d[k]).wait()
        pltpu.async_copy(drows.at[k], df_hbm.at[pl.ds(base + off, GCH)],
                         wsem_d[k])
      return carry

    lax.fori_loop(0, NGC // 2, body, 0)
    for k in range(2):
      pltpu.make_async_copy(
          srows.at[k], sf_hbm.at[pl.ds(base, GCH)], wsem_s[k]).wait()
      pltpu.make_async_copy(
          drows.at[k], df_hbm.at[pl.ds(base, GCH)], wsem_d[k]).wait()

  return k(xpad, src2d, dst2d)


def _sc_scatter(msgs, dst2d, zrows):
  @functools.partial(
      pl.kernel,
      out_type=jax.ShapeDtypeStruct((NCORES, NACC, H), jnp.float32),
      mesh=_sc_mesh(),
      scratch_types=[pltpu.VMEM((NCH, CHUNK), jnp.int32),
                     pltpu.VMEM((CHUNK, H), jnp.float32),
                     pltpu.VMEM_SHARED((NACC, H), jnp.float32)],
  )
  def k(msgs_hbm, dst_hbm, z_hbm, out_hbm, didx, rows, acc):
    c = lax.axis_index("c")
    s = lax.axis_index("s")
    wid = s * NCORES + c
    pltpu.sync_copy(z_hbm, rows)

    def zbody(i, carry):
      pltpu.sync_copy(rows, acc.at[pl.ds(s * RPT + i * CHUNK, CHUNK)])
      return carry

    lax.fori_loop(0, RPT // CHUNK, zbody, 0)
    plsc.subcore_barrier()

    pltpu.sync_copy(dst_hbm.at[pl.ds(wid * NCH, NCH)], didx)
    ebase = wid * EPW

    def body(j, carry):
      pltpu.sync_copy(msgs_hbm.at[pl.ds(ebase + j * CHUNK, CHUNK)], rows)
      pltpu.sync_copy(rows, acc.at[didx.at[j]], add=True)
      return carry

    lax.fori_loop(0, NCH, body, 0)
    plsc.subcore_barrier()

    def wbody(i, carry):
      off = s * RPT + i * CHUNK
      pltpu.sync_copy(acc.at[pl.ds(off, CHUNK)], rows)
      pltpu.sync_copy(rows, out_hbm.at[c].at[pl.ds(off, CHUNK)])
      return carry

    lax.fori_loop(0, RPT // CHUNK, wbody, 0)

  return k(msgs, dst2d, zrows)


def _embed_body(an_ref, tab_ref, out_ref):
  oh = (an_ref[...] == lax.broadcasted_iota(jnp.int32, (1, 128), 1))
  out_ref[...] = jnp.dot(oh.astype(jnp.float32), tab_ref[...],
                         preferred_element_type=jnp.float32)


def _tc_embed(an2d, tab):
  return pl.pallas_call(
      _embed_body,
      out_shape=jax.ShapeDtypeStruct((N, H), jnp.float32),
  )(an2d, tab)


def _edge_body(sf_ref, df_ref, ef_ref, ws_ref, wd_ref, wef_ref, bf_ref,
               w2_ref, b2_ref, out_ref):
  sfb = sf_ref[...]
  h = jnp.dot(sfb, ws_ref[...], preferred_element_type=jnp.float32)
  h = h + jnp.dot(df_ref[...], wd_ref[...], preferred_element_type=jnp.float32)
  h = h + jnp.dot(ef_ref[...], wef_ref[...], preferred_element_type=jnp.float32)
  h = jax.nn.relu(h + bf_ref[...])
  ew = jnp.dot(h, w2_ref[...], preferred_element_type=jnp.float32) + b2_ref[...]
  out_ref[...] = sfb * jax.nn.sigmoid(ew)


def _tc_edge_mlp(sf, df, efp, ws, wd, wef, bf, w2, b2):
  nb = EPAD // BE
  blk = lambda i: (i, 0)
  fix = lambda i: (0, 0)
  return pl.pallas_call(
      _edge_body,
      grid=(nb,),
      in_specs=[pl.BlockSpec((BE, H), blk),
                pl.BlockSpec((BE, H), blk),
                pl.BlockSpec((BE, 16), blk),
                pl.BlockSpec((H, H), fix),
                pl.BlockSpec((H, H), fix),
                pl.BlockSpec((16, H), fix),
                pl.BlockSpec((1, H), fix),
                pl.BlockSpec((H, H), fix),
                pl.BlockSpec((1, H), fix)],
      out_specs=pl.BlockSpec((BE, H), blk),
      out_shape=jax.ShapeDtypeStruct((EPAD, H), jnp.float32),
  )(sf, df, efp, ws, wd, wef, bf, w2, b2)


def _node_body(x_ref, a_ref, w1a_ref, w1b_ref, b1_ref, w2_ref, b2_ref,
               g_ref, bb_ref, out_ref):
  x = x_ref[...]
  a = a_ref[0, :N, :] + a_ref[1, :N, :]
  t = jnp.dot(x, w1a_ref[...], preferred_element_type=jnp.float32)
  t = t + jnp.dot(a, w1b_ref[...], preferred_element_type=jnp.float32)
  t = jax.nn.relu(t + b1_ref[...])
  u = jnp.dot(t, w2_ref[...], preferred_element_type=jnp.float32) + b2_ref[...]
  mu = jnp.mean(u, axis=0, keepdims=True)
  d = u - mu
  var = jnp.mean(d * d, axis=0, keepdims=True)
  un = d / jnp.sqrt(var + 1e-5) * g_ref[...] + bb_ref[...]
  out_ref[...] = x + un


def _tc_node(x, agg2, w1a, w1b, b1, w2, b2, g, bb):
  return pl.pallas_call(
      _node_body,
      out_shape=jax.ShapeDtypeStruct((N, H), jnp.float32),
  )(x, agg2, w1a, w1b, b1, w2, b2, g, bb)


def _readout_body(x_ref, bt_ref, w1_ref, b1_ref, w2_ref, b2_ref, out_ref):
  bt = bt_ref[...]
  oh = (bt == lax.broadcasted_iota(jnp.int32, (B, 1), 0)).astype(jnp.float32)
  pooled = jnp.dot(oh, x_ref[...], preferred_element_type=jnp.float32)
  counts = jnp.sum(oh, axis=1, keepdims=True)
  pooled = pooled / jnp.clip(counts, 1.0, None)
  t = jax.nn.relu(jnp.dot(pooled, w1_ref[...],
                          preferred_element_type=jnp.float32) + b1_ref[...])
  emb = jnp.dot(t, w2_ref[...], preferred_element_type=jnp.float32) + b2_ref[...]
  nrm = jnp.sqrt(jnp.sum(emb * emb, axis=1, keepdims=True))
  out_ref[...] = emb / jnp.maximum(nrm, 1e-12)


def _tc_readout(x, brow, w1, b1, w2, b2):
  return pl.pallas_call(
      _readout_body,
      out_shape=jax.ShapeDtypeStruct((B, H), jnp.float32),
  )(x, brow, w1, b1, w2, b2)


def kernel(atomic_numbers, edge_index, edge_features, batch, params):
  f32 = jnp.float32
  src = edge_index[0].astype(jnp.int32)
  dst = edge_index[1].astype(jnp.int32)
  srcp = jnp.pad(src, (0, EPAD - E))
  dstp = jnp.pad(dst, (0, EPAD - E), constant_values=N)
  dst2d = dstp.reshape(NW * NCH, CHUNK)
  src2g = srcp.reshape(NW * NGC, GCH)
  dst2g = dstp.reshape(NW * NGC, GCH)
  efp = jnp.pad(edge_features.astype(f32), ((0, EPAD - E), (0, 16 - ED)))
  tab = jnp.pad(params["elem_table"].astype(f32), ((0, 128 - NELEM), (0, 0)))
  an2d = (atomic_numbers.astype(jnp.int32) - 1).reshape(N, 1)
  zrows = jnp.zeros((CHUNK, H), f32)

  x = _tc_embed(an2d, tab)

  wep = jnp.pad(params["edge_embed"]["w"].astype(f32), ((0, 16 - ED), (0, 0)))
  be = params["edge_embed"]["b"].astype(f32)

  for c in params["convs"]:
    w1 = c["edge1"]["w"].astype(f32)
    ws, wd, w1e = w1[:H], w1[H:2 * H], w1[2 * H:]
    wef = wep @ w1e
    bf = (be @ w1e + c["edge1"]["b"]).reshape(1, H)
    xpad = jnp.pad(x, ((0, NACC - N), (0, 0)))
    sf, df = _sc_gather(xpad, src2g, dst2g)
    msgs = _tc_edge_mlp(sf, df, efp, ws, wd, wef, bf,
                        c["edge2"]["w"].astype(f32),
                        c["edge2"]["b"].reshape(1, H).astype(f32))
    agg2 = _sc_scatter(msgs, dst2d, zrows)
    x = _tc_node(x, agg2,
                 c["node1"]["w"][:H].astype(f32),
                 c["node1"]["w"][H:].astype(f32),
                 c["node1"]["b"].reshape(1, H).astype(f32),
                 c["node2"]["w"].astype(f32),
                 c["node2"]["b"].reshape(1, H).astype(f32),
                 c["bn_g"].reshape(1, H).astype(f32),
                 c["bn_b"].reshape(1, H).astype(f32))

  brow = batch.astype(jnp.int32).reshape(1, N)
  return _tc_readout(x, brow,
                     params["readout1"]["w"].astype(f32),
                     params["readout1"]["b"].reshape(1, H).astype(f32),
                     params["readout2"]["w"].astype(f32),
                     params["readout2"]["b"].reshape(1, H).astype(f32))


# speedup vs baseline: 4.0652x; 1.0902x over previous
import functools

import jax
import jax.numpy as jnp
from jax import lax
from jax.experimental import pallas as pl
from jax.experimental.pallas import tpu as pltpu
from jax.experimental.pallas import tpu_sc as plsc

N = 10000
E = 320000
H = 128
ED = 10
B = 64
NELEM = 92

NCORES = 2
NSUB = 16
NW = NCORES * NSUB
CHUNK = 128
EPW = 10240
EPAD = NW * EPW
NCH = EPW // CHUNK
NACC = 10240
RPT = NACC // NSUB
BE = 2048
GCH = 64
NGC = EPW // GCH
IB = 16
NSL = 2
EPWS = EPW // NSL
EPADS = EPAD // NSL
NGCS = NGC // NSL
NCHS = NCH // NSL

@functools.cache
def _sc_mesh():
  return plsc.VectorSubcoreMesh(
      core_axis_name="c", subcore_axis_name="s",
      num_cores=NCORES, num_subcores=NSUB)


def _sc_gather(xpad, src2d, dst2d):
  @functools.partial(
      pl.kernel,
      out_type=[jax.ShapeDtypeStruct((EPADS, H), jnp.float32),
                jax.ShapeDtypeStruct((EPADS, H), jnp.float32)],
      mesh=_sc_mesh(),
      scratch_types=[pltpu.VMEM((IB, GCH), jnp.int32),
                     pltpu.VMEM((IB, GCH), jnp.int32),
                     pltpu.VMEM((2, GCH, H), jnp.float32),
                     pltpu.VMEM((2, GCH, H), jnp.float32),
                     pltpu.VMEM_SHARED((NACC, H), jnp.float32),
                     [pltpu.SemaphoreType.DMA] * 2,
                     [pltpu.SemaphoreType.DMA] * 2,
                     [pltpu.SemaphoreType.DMA] * 2,
                     [pltpu.SemaphoreType.DMA] * 2],
  )
  def k(x_hbm, src_hbm, dst_hbm, sf_hbm, df_hbm, sidx, didx, srows, drows,
        xsh, gsem_s, gsem_d, wsem_s, wsem_d):
    s = lax.axis_index("s")
    wid = s * NCORES + lax.axis_index("c")
    base = wid * EPWS
    ibase = wid * NGCS

    pltpu.sync_copy(x_hbm.at[pl.ds(s * RPT, RPT)], xsh.at[pl.ds(s * RPT, RPT)])
    plsc.subcore_barrier()

    def body(g, carry):
      @pl.when(g % (IB // 2) == 0)
      def _reload():
        blk = g // (IB // 2)
        pltpu.sync_copy(src_hbm.at[pl.ds(ibase + blk * IB, IB)], sidx)
        pltpu.sync_copy(dst_hbm.at[pl.ds(ibase + blk * IB, IB)], didx)

      r0 = (2 * g) % IB
      for k in range(2):
        @pl.when(g > 0)
        def _drain():
          pltpu.make_async_copy(
              srows.at[k], sf_hbm.at[pl.ds(base, GCH)], wsem_s[k]).wait()
          pltpu.make_async_copy(
              drows.at[k], df_hbm.at[pl.ds(base, GCH)], wsem_d[k]).wait()

        pltpu.async_copy(xsh.at[sidx.at[r0 + k]], srows.at[k], gsem_s[k])
        pltpu.async_copy(xsh.at[didx.at[r0 + k]], drows.at[k], gsem_d[k])
      for k in range(2):
        off = (2 * g + k) * GCH
        pltpu.make_async_copy(xsh.at[sidx.at[r0 + k]],
                              srows.at[k], gsem_s[k]).wait()
        pltpu.async_copy(srows.at[k], sf_hbm.at[pl.ds(base + off, GCH)],
                         wsem_s[k])
        pltpu.make_async_copy(xsh.at[didx.at[r0 + k]],
                              drows.at[k], gsem_d[k]).wait()
        pltpu.async_copy(drows.at[k], df_hbm.at[pl.ds(base + off, GCH)],
                         wsem_d[k])
      return carry

    lax.fori_loop(0, NGCS // 2, body, 0)
    for k in range(2):
      pltpu.make_async_copy(
          srows.at[k], sf_hbm.at[pl.ds(base, GCH)], wsem_s[k]).wait()
      pltpu.make_async_copy(
          drows.at[k], df_hbm.at[pl.ds(base, GCH)], wsem_d[k]).wait()

  return k(xpad, src2d, dst2d)


def _sc_scatter(msgs, dst2d, zrows):
  @functools.partial(
      pl.kernel,
      out_type=jax.ShapeDtypeStruct((NCORES, NACC, H), jnp.float32),
      mesh=_sc_mesh(),
      scratch_types=[pltpu.VMEM((NCHS, CHUNK), jnp.int32),
                     pltpu.VMEM((CHUNK, H), jnp.float32),
                     pltpu.VMEM_SHARED((NACC, H), jnp.float32)],
  )
  def k(msgs_hbm, dst_hbm, z_hbm, out_hbm, didx, rows, acc):
    c = lax.axis_index("c")
    s = lax.axis_index("s")
    wid = s * NCORES + c
    pltpu.sync_copy(z_hbm, rows)

    def zbody(i, carry):
      pltpu.sync_copy(rows, acc.at[pl.ds(s * RPT + i * CHUNK, CHUNK)])
      return carry

    lax.fori_loop(0, RPT // CHUNK, zbody, 0)
    plsc.subcore_barrier()

    pltpu.sync_copy(dst_hbm.at[pl.ds(wid * NCHS, NCHS)], didx)
    ebase = wid * EPWS

    def body(j, carry):
      pltpu.sync_copy(msgs_hbm.at[pl.ds(ebase + j * CHUNK, CHUNK)], rows)
      pltpu.sync_copy(rows, acc.at[didx.at[j]], add=True)
      return carry

    lax.fori_loop(0, NCHS, body, 0)
    plsc.subcore_barrier()

    def wbody(i, carry):
      off = s * RPT + i * CHUNK
      pltpu.sync_copy(acc.at[pl.ds(off, CHUNK)], rows)
      pltpu.sync_copy(rows, out_hbm.at[c].at[pl.ds(off, CHUNK)])
      return carry

    lax.fori_loop(0, RPT // CHUNK, wbody, 0)

  return k(msgs, dst2d, zrows)


def _embed_body(an_ref, tab_ref, out_ref):
  oh = (an_ref[...] == lax.broadcasted_iota(jnp.int32, (1, 128), 1))
  out_ref[...] = jnp.dot(oh.astype(jnp.float32), tab_ref[...],
                         preferred_element_type=jnp.float32)


def _tc_embed(an2d, tab):
  return pl.pallas_call(
      _embed_body,
      out_shape=jax.ShapeDtypeStruct((N, H), jnp.float32),
  )(an2d, tab)


def _edge_body(sf_ref, df_ref, ef_ref, ws_ref, wd_ref, wef_ref, bf_ref,
               w2_ref, b2_ref, out_ref):
  sfb = sf_ref[...]
  h = jnp.dot(sfb, ws_ref[...], preferred_element_type=jnp.float32)
  h = h + jnp.dot(df_ref[...], wd_ref[...], preferred_element_type=jnp.float32)
  h = h + jnp.dot(ef_ref[...], wef_ref[...], preferred_element_type=jnp.float32)
  h = jax.nn.relu(h + bf_ref[...])
  ew = jnp.dot(h, w2_ref[...], preferred_element_type=jnp.float32) + b2_ref[...]
  out_ref[...] = sfb * jax.nn.sigmoid(ew)


def _tc_edge_mlp(sf, df, efp, ws, wd, wef, bf, w2, b2):
  nb = EPADS // BE
  blk = lambda i: (i, 0)
  fix = lambda i: (0, 0)
  return pl.pallas_call(
      _edge_body,
      grid=(nb,),
      in_specs=[pl.BlockSpec((BE, H), blk),
                pl.BlockSpec((BE, H), blk),
                pl.BlockSpec((BE, 16), blk),
                pl.BlockSpec((H, H), fix),
                pl.BlockSpec((H, H), fix),
                pl.BlockSpec((16, H), fix),
                pl.BlockSpec((1, H), fix),
                pl.BlockSpec((H, H), fix),
                pl.BlockSpec((1, H), fix)],
      out_specs=pl.BlockSpec((BE, H), blk),
      out_shape=jax.ShapeDtypeStruct((EPADS, H), jnp.float32),
  )(sf, df, efp, ws, wd, wef, bf, w2, b2)


def _node_body(x_ref, a0_ref, a1_ref, w1a_ref, w1b_ref, b1_ref, w2_ref,
               b2_ref, g_ref, bb_ref, out_ref):
  x = x_ref[...]
  a = (a0_ref[0, :N, :] + a0_ref[1, :N, :]
       + a1_ref[0, :N, :] + a1_ref[1, :N, :])
  t = jnp.dot(x, w1a_ref[...], preferred_element_type=jnp.float32)
  t = t + jnp.dot(a, w1b_ref[...], preferred_element_type=jnp.float32)
  t = jax.nn.relu(t + b1_ref[...])
  u = jnp.dot(t, w2_ref[...], preferred_element_type=jnp.float32) + b2_ref[...]
  mu = jnp.mean(u, axis=0, keepdims=True)
  d = u - mu
  var = jnp.mean(d * d, axis=0, keepdims=True)
  un = d / jnp.sqrt(var + 1e-5) * g_ref[...] + bb_ref[...]
  out_ref[...] = x + un


def _tc_node(x, agg_a, agg_b, w1a, w1b, b1, w2, b2, g, bb):
  return pl.pallas_call(
      _node_body,
      out_shape=jax.ShapeDtypeStruct((N, H), jnp.float32),
  )(x, agg_a, agg_b, w1a, w1b, b1, w2, b2, g, bb)


def _readout_body(x_ref, bt_ref, w1_ref, b1_ref, w2_ref, b2_ref, out_ref):
  bt = bt_ref[...]
  oh = (bt == lax.broadcasted_iota(jnp.int32, (B, 1), 0)).astype(jnp.float32)
  pooled = jnp.dot(oh, x_ref[...], preferred_element_type=jnp.float32)
  counts = jnp.sum(oh, axis=1, keepdims=True)
  pooled = pooled / jnp.clip(counts, 1.0, None)
  t = jax.nn.relu(jnp.dot(pooled, w1_ref[...],
                          preferred_element_type=jnp.float32) + b1_ref[...])
  emb = jnp.dot(t, w2_ref[...], preferred_element_type=jnp.float32) + b2_ref[...]
  nrm = jnp.sqrt(jnp.sum(emb * emb, axis=1, keepdims=True))
  out_ref[...] = emb / jnp.maximum(nrm, 1e-12)


def _tc_readout(x, brow, w1, b1, w2, b2):
  return pl.pallas_call(
      _readout_body,
      out_shape=jax.ShapeDtypeStruct((B, H), jnp.float32),
  )(x, brow, w1, b1, w2, b2)


def kernel(atomic_numbers, edge_index, edge_features, batch, params):
  f32 = jnp.float32
  src = edge_index[0].astype(jnp.int32)
  dst = edge_index[1].astype(jnp.int32)
  srcp = jnp.pad(src, (0, EPAD - E))
  dstp = jnp.pad(dst, (0, EPAD - E), constant_values=N)
  dst2d = dstp.reshape(NSL, NW * NCHS, CHUNK)
  src2g = srcp.reshape(NSL, NW * NGCS, GCH)
  dst2g = dstp.reshape(NSL, NW * NGCS, GCH)
  efp = jnp.pad(edge_features.astype(f32), ((0, EPAD - E), (0, 16 - ED)))
  efsl = efp.reshape(NSL, EPADS, 16)
  tab = jnp.pad(params["elem_table"].astype(f32), ((0, 128 - NELEM), (0, 0)))
  an2d = (atomic_numbers.astype(jnp.int32) - 1).reshape(N, 1)
  zrows = jnp.zeros((CHUNK, H), f32)

  x = _tc_embed(an2d, tab)

  wep = jnp.pad(params["edge_embed"]["w"].astype(f32), ((0, 16 - ED), (0, 0)))
  be = params["edge_embed"]["b"].astype(f32)

  for c in params["convs"]:
    w1 = c["edge1"]["w"].astype(f32)
    ws, wd, w1e = w1[:H], w1[H:2 * H], w1[2 * H:]
    wef = wep @ w1e
    bf = (be @ w1e + c["edge1"]["b"]).reshape(1, H)
    xpad = jnp.pad(x, ((0, NACC - N), (0, 0)))
    aggs = []
    for sl in range(NSL):
      sf, df = _sc_gather(xpad, src2g[sl], dst2g[sl])
      msgs = _tc_edge_mlp(sf, df, efsl[sl], ws, wd, wef, bf,
                          c["edge2"]["w"].astype(f32),
                          c["edge2"]["b"].reshape(1, H).astype(f32))
      aggs.append(_sc_scatter(msgs, dst2d[sl], zrows))
    x = _tc_node(x, aggs[0], aggs[1],
                 c["node1"]["w"][:H].astype(f32),
                 c["node1"]["w"][H:].astype(f32),
                 c["node1"]["b"].reshape(1, H).astype(f32),
                 c["node2"]["w"].astype(f32),
                 c["node2"]["b"].reshape(1, H).astype(f32),
                 c["bn_g"].reshape(1, H).astype(f32),
                 c["bn_b"].reshape(1, H).astype(f32))

  brow = batch.astype(jnp.int32).reshape(1, N)
  return _tc_readout(x, brow,
                     params["readout1"]["w"].astype(f32),
                     params["readout1"]["b"].reshape(1, H).astype(f32),
                     params["readout2"]["w"].astype(f32),
                     params["readout2"]["b"].reshape(1, H).astype(f32))
